# baseline (speedup 1.0000x reference)
import jax
import jax.numpy as jnp
from jax.experimental import pallas as pl


def _bias_add(y_ref, b_ref, o_ref):
    o_ref[...] = y_ref[...] + b_ref[...]


def kernel(X, values, bias, crow_indices, col_indices):
    nnz = values.shape[0]
    n_rows = crow_indices.shape[0] - 1
    row_ids = jnp.searchsorted(crow_indices,
                               jnp.arange(nnz, dtype=crow_indices.dtype),
                               side="right") - 1
    gathered = jnp.take(X, col_indices, axis=1) * values[None, :]
    yT = jax.ops.segment_sum(gathered.T, row_ids, num_segments=n_rows)
    y = yT.T
    return pl.pallas_call(
        _bias_add,
        out_shape=jax.ShapeDtypeStruct(y.shape, y.dtype),
    )(y, jnp.broadcast_to(bias[None, :], y.shape))


# speedup vs baseline: 147.5087x; 147.5087x over previous
import functools

import jax
import jax.numpy as jnp
from jax import lax
from jax.experimental import pallas as pl
from jax.experimental.pallas import tpu as pltpu
from jax.experimental.pallas import tpu_sc as plsc

N = 16384
B = 64
NNZ_PER_ROW = 16
CH = 16
CHN = CH * NNZ_PER_ROW


def _make_kernel():
    info = plsc.get_sparse_core_info()
    nc, ns = info.num_cores, info.num_subcores
    nw = nc * ns
    rows_per_w = N // nw
    n_chunks = rows_per_w // CH

    mesh = plsc.VectorSubcoreMesh(core_axis_name="c", subcore_axis_name="s")

    @functools.partial(
        pl.kernel,
        out_type=jax.ShapeDtypeStruct((N, B), jnp.float32),
        mesh=mesh,
        compiler_params=pltpu.CompilerParams(use_tc_tiling_on_sc=False),
        scratch_types=[
            pltpu.VMEM((128,), jnp.int32),
            pltpu.VMEM((128,), jnp.int32),
            pltpu.VMEM((CHN,), jnp.float32),
            pltpu.VMEM((CH,), jnp.float32),
            pltpu.VMEM((CHN, B), jnp.float32),
            pltpu.VMEM((CH, B), jnp.float32),
            pltpu.SemaphoreType.DMA,
        ],
    )
    def k(xt_hbm, col_hbm, val_hbm, bias_hbm, out_hbm,
          idx_a, idx_b, val_v, bias_v, gbuf, obuf, sem):
        wid = lax.axis_index("s") * nc + lax.axis_index("c")
        row0 = wid * rows_per_w

        def chunk(t, _):
            r0 = row0 + t * CH
            nz0 = r0 * NNZ_PER_ROW
            pltpu.sync_copy(col_hbm.at[pl.ds(nz0, 128)], idx_a)
            pltpu.sync_copy(col_hbm.at[pl.ds(nz0 + 128, 128)], idx_b)
            pltpu.sync_copy(val_hbm.at[pl.ds(nz0, CHN)], val_v)
            pltpu.sync_copy(bias_hbm.at[pl.ds(r0, CH)], bias_v)
            ga = pltpu.async_copy(xt_hbm.at[idx_a], gbuf.at[pl.ds(0, 128)], sem)
            gb = pltpu.async_copy(xt_hbm.at[idx_b], gbuf.at[pl.ds(128, 128)], sem)
            ga.wait()
            gb.wait()
            bv = bias_v[pl.ds(0, CH)]
            for i in range(CH):
                vv = val_v[pl.ds(i * NNZ_PER_ROW, 16)]
                bb = bv[i]
                accs = [jnp.full((16,), 0.0, jnp.float32) + bb for _ in range(4)]
                for j in range(NNZ_PER_ROW):
                    w = vv[j]
                    g = i * NNZ_PER_ROW + j
                    for c in range(4):
                        accs[c] = accs[c] + w * gbuf[g, pl.ds(c * 16, 16)]
                for c in range(4):
                    obuf[i, pl.ds(c * 16, 16)] = accs[c]
            pltpu.sync_copy(obuf, out_hbm.at[pl.ds(r0, CH)])
            return ()

        lax.fori_loop(0, n_chunks, chunk, ())

    return k


def kernel(X, values, bias, crow_indices, col_indices):
    del crow_indices
    xt = X.T.reshape(N, B)
    yt = _make_kernel()(xt, col_indices, values, bias)
    return yt.T.reshape(B, N)


# speedup vs baseline: 305.5677x; 2.0715x over previous
import functools

import jax
import jax.numpy as jnp
from jax import lax
from jax.experimental import pallas as pl
from jax.experimental.pallas import tpu as pltpu
from jax.experimental.pallas import tpu_sc as plsc

N = 16384
B = 64
NNZ_PER_ROW = 16
CH = 16
CHN = CH * NNZ_PER_ROW


def _make_kernel():
    info = plsc.get_sparse_core_info()
    nc, ns = info.num_cores, info.num_subcores
    nw = nc * ns
    rows_per_w = N // nw
    n_chunks = rows_per_w // CH
    halves_per_w = 2 * n_chunks

    mesh = plsc.VectorSubcoreMesh(core_axis_name="c", subcore_axis_name="s")

    @functools.partial(
        pl.kernel,
        out_type=jax.ShapeDtypeStruct((N, B), jnp.float32),
        mesh=mesh,
        compiler_params=pltpu.CompilerParams(use_tc_tiling_on_sc=False),
        scratch_types=[
            pltpu.VMEM((halves_per_w, 128), jnp.int32),
            pltpu.VMEM((rows_per_w * NNZ_PER_ROW,), jnp.float32),
            pltpu.VMEM((rows_per_w,), jnp.float32),
            pltpu.VMEM((CHN, B), jnp.float32),
            pltpu.VMEM((CHN, B), jnp.float32),
            pltpu.VMEM((CH, B), jnp.float32),
            pltpu.VMEM((CH, B), jnp.float32),
            pltpu.SemaphoreType.DMA,
            pltpu.SemaphoreType.DMA,
            pltpu.SemaphoreType.DMA,
            pltpu.SemaphoreType.DMA,
        ],
    )
    def k(xt_hbm, col_hbm, val_hbm, bias_hbm, out_hbm,
          col_v, val_v, bias_v, gba, gbb, oba, obb, ga, gb, sa, sb):
        wid = lax.axis_index("s") * nc + lax.axis_index("c")
        row0 = wid * rows_per_w

        pltpu.sync_copy(col_hbm.at[pl.ds(wid * halves_per_w, halves_per_w)],
                        col_v)
        pltpu.sync_copy(val_hbm.at[pl.ds(row0 * NNZ_PER_ROW,
                                         rows_per_w * NNZ_PER_ROW)], val_v)
        pltpu.sync_copy(bias_hbm.at[pl.ds(row0, rows_per_w)], bias_v)

        def fire_gather(t, gbuf, sem):
            pltpu.async_copy(xt_hbm.at[col_v.at[2 * t]],
                             gbuf.at[pl.ds(0, 128)], sem)
            pltpu.async_copy(xt_hbm.at[col_v.at[2 * t + 1]],
                             gbuf.at[pl.ds(128, 128)], sem)

        def wait_gather(gbuf, sem):
            pltpu.make_async_copy(xt_hbm.at[pl.ds(0, 128)],
                                  gbuf.at[pl.ds(0, 128)], sem).wait()
            pltpu.make_async_copy(xt_hbm.at[pl.ds(0, 128)],
                                  gbuf.at[pl.ds(128, 128)], sem).wait()

        def wait_store(obuf, sem):
            pltpu.make_async_copy(obuf, out_hbm.at[pl.ds(0, CH)], sem).wait()

        def compute(t, gbuf, obuf, sem):
            bv = bias_v[pl.ds(t * CH, CH)]
            for i in range(CH):
                vv = val_v[pl.ds((t * CH + i) * NNZ_PER_ROW, 16)]
                bb = bv[i]
                accs = [jnp.full((16,), 0.0, jnp.float32) + bb
                        for _ in range(4)]
                for j in range(NNZ_PER_ROW):
                    w = vv[j]
                    g = i * NNZ_PER_ROW + j
                    for c in range(4):
                        accs[c] = accs[c] + w * gbuf[g, pl.ds(c * 16, 16)]
                for c in range(4):
                    obuf[i, pl.ds(c * 16, 16)] = accs[c]
            pltpu.async_copy(obuf, out_hbm.at[pl.ds(row0 + t * CH, CH)], sem)

        fire_gather(0, gba, ga)

        def body(tt, _):
            t0 = 2 * tt
            t1 = t0 + 1
            fire_gather(t1, gbb, gb)
            wait_gather(gba, ga)

            @pl.when(tt > 0)
            def _():
                wait_store(oba, sa)

            compute(t0, gba, oba, sa)

            @pl.when(tt < n_chunks // 2 - 1)
            def _():
                fire_gather(t0 + 2, gba, ga)

            wait_gather(gbb, gb)

            @pl.when(tt > 0)
            def _():
                wait_store(obb, sb)

            compute(t1, gbb, obb, sb)
            return ()

        lax.fori_loop(0, n_chunks // 2, body, ())
        wait_store(oba, sa)
        wait_store(obb, sb)

    return k


def kernel(X, values, bias, crow_indices, col_indices):
    del crow_indices
    xt = X.T.reshape(N, B)
    col2d = col_indices.reshape(-1, 128)
    yt = _make_kernel()(xt, col2d, values, bias)
    return yt.T.reshape(B, N)
